# baseline (speedup 1.0000x reference)
import functools

import jax
import jax.numpy as jnp
from jax import lax
from jax.experimental import pallas as pl
from jax.experimental.pallas import tpu as pltpu
from jax.experimental.pallas import tpu_sc as plsc


_NC, _NS = 2, 16
_NW = _NC * _NS
_LANE = 128
_TROW = 8


def _detile_body(x_ref, o_ref):
    m, n_pk, _ = o_ref.shape
    for j in range(m):
        x = x_ref[:, j * _LANE:(j + 1) * _LANE]
        for p in range(n_pk // 4):
            lo = lax.bitcast_convert_type(
                x[8 * p:8 * p + 4, :].astype(jnp.bfloat16),
                jnp.uint16).astype(jnp.uint32)
            hi = lax.bitcast_convert_type(
                x[8 * p + 4:8 * p + 8, :].astype(jnp.bfloat16),
                jnp.uint16).astype(jnp.uint32)
            o_ref[j, pl.ds(4 * p, 4), :] = lax.bitcast_convert_type(
                lo | (hi << 16), jnp.int32)


@functools.lru_cache(maxsize=None)
def _make_detile(emb_dim, vocab):
    n_tiles = pl.cdiv(vocab, _LANE)
    m = 256
    n_jb = pl.cdiv(n_tiles, m)
    return pl.pallas_call(
        _detile_body,
        grid=(n_jb,),
        in_specs=[pl.BlockSpec((emb_dim, _LANE * m), lambda jb: (0, jb))],
        out_specs=pl.BlockSpec((m, emb_dim // 2, _LANE),
                               lambda jb: (jb, 0, 0)),
        out_shape=jax.ShapeDtypeStruct((n_tiles, emb_dim // 2, _LANE),
                                       jnp.int32),
    )


@functools.lru_cache(maxsize=None)
def _make_gather(batch, n_pk, flat_len):
    b_per_w = batch // _NW
    n_q = b_per_w // _LANE
    mesh = plsc.VectorSubcoreMesh(core_axis_name="c", subcore_axis_name="s")
    out_sh = jax.ShapeDtypeStruct((n_pk, batch), jnp.int32)

    def build_flat_idx(idx_v, fidx_v):
        for q in range(n_q):
            for s in range(_LANE // 16):
                r = idx_v[q, pl.ds(s * 16, 16)]
                base = (r >> 7) * (n_pk * _LANE) + (r & (_LANE - 1))
                for c in range(n_pk):
                    fidx_v[c, q, pl.ds(s * 16, 16)] = base + _LANE * c

    @functools.partial(
        pl.kernel,
        mesh=mesh,
        out_type=out_sh,
        scratch_types=[
            pltpu.VMEM((n_q, _LANE), jnp.int32),
            pltpu.VMEM((n_pk, n_q, _LANE), jnp.int32),
            pltpu.VMEM((n_pk, b_per_w), jnp.int32),
            pltpu.SemaphoreType.DMA,
        ],
        compiler_params=pltpu.CompilerParams(use_tc_tiling_on_sc=False),
    )
    def gather_kernel(idx_hbm, tab_hbm, out_hbm, idx_v, fidx_v, rows_v, sem):
        wid = lax.axis_index("s") * _NC + lax.axis_index("c")
        base = wid * b_per_w
        for q in range(n_q):
            pltpu.sync_copy(idx_hbm.at[pl.ds(base + q * _LANE, _LANE)],
                            idx_v.at[q])
        build_flat_idx(idx_v, fidx_v)
        copies = []
        for c in range(n_pk):
            for q in range(n_q):
                copies.append(pltpu.async_copy(
                    tab_hbm.at[fidx_v.at[c, q]],
                    rows_v.at[c, pl.ds(q * _LANE, _LANE)], sem))
        for cp in copies:
            cp.wait()
        pltpu.sync_copy(rows_v, out_hbm.at[:, pl.ds(base, b_per_w)])

    return gather_kernel


def _unpack(x):
    xu = lax.bitcast_convert_type(x, jnp.uint32)
    lo = lax.bitcast_convert_type(xu << 16, jnp.float32)
    hi = lax.bitcast_convert_type(xu & jnp.uint32(0xFFFF0000), jnp.float32)
    return lo, hi


def _mlp_body(ut_ref, vt_ref, w1_ref, b1_ref, w2_ref, b2_ref, w3_ref, b3_ref,
              o_ref):
    n_pk = ut_ref.shape[0]
    ulo, uhi = _unpack(ut_ref[...])
    vlo, vhi = _unpack(vt_ref[...])
    dn = (((0,), (0,)), ((), ()))
    h = lax.dot_general(w1_ref[0:n_pk, :], ulo, dn,
                        preferred_element_type=jnp.float32)
    h = h + lax.dot_general(w1_ref[n_pk:2 * n_pk, :], uhi, dn,
                            preferred_element_type=jnp.float32)
    h = h + lax.dot_general(w1_ref[2 * n_pk:3 * n_pk, :], vlo, dn,
                            preferred_element_type=jnp.float32)
    h = h + lax.dot_general(w1_ref[3 * n_pk:, :], vhi, dn,
                            preferred_element_type=jnp.float32)
    h = jnp.maximum(h + b1_ref[...][:, None], 0.0)
    h = lax.dot_general(w2_ref[...], h, dn,
                        preferred_element_type=jnp.float32)
    h = jnp.maximum(h + b2_ref[...][:, None], 0.0)
    w3 = w3_ref[...]
    o_ref[...] = jnp.sum(h * w3, axis=0) + b3_ref[...]


def kernel(user, item, user_emb, item_emb, W1, b1, W2, b2, W3, b3):
    batch = user.shape[0]
    vocab, emb_dim = user_emb.shape
    n_pk = emb_dim // 2
    detile = _make_detile(emb_dim, vocab)
    u_tab = detile(user_emb.T)
    v_tab = detile(item_emb.T)
    flat_len = u_tab.shape[0] * n_pk * _LANE
    gather = _make_gather(batch, n_pk, flat_len)
    ut_g = gather(user, u_tab.reshape(flat_len))
    vt_g = gather(item, v_tab.reshape(flat_len))
    perm_lo = [8 * (c2 // 4) + c2 % 4 for c2 in range(n_pk)]
    perm = (perm_lo + [p + 4 for p in perm_lo]
            + [p + emb_dim for p in perm_lo]
            + [p + emb_dim + 4 for p in perm_lo])
    w1p = W1[jnp.asarray(perm, dtype=jnp.int32), :]
    return pl.pallas_call(
        _mlp_body,
        out_shape=jax.ShapeDtypeStruct((batch,), jnp.float32),
    )(ut_g, vt_g, w1p, b1, W2, b2, W3, b3)


# speedup vs baseline: 1.0519x; 1.0519x over previous
import functools

import jax
import jax.numpy as jnp
from jax import lax
from jax.experimental import pallas as pl
from jax.experimental.pallas import tpu as pltpu
from jax.experimental.pallas import tpu_sc as plsc


_NC, _NS = 2, 16
_NW = _NC * _NS
_LANE = 128
_TROW = 8


def _detile_body(x_ref, o_ref):
    m, n_pk, _ = o_ref.shape
    for j in range(m):
        x = x_ref[:, j * _LANE:(j + 1) * _LANE]
        for p in range(n_pk // 4):
            lo = lax.bitcast_convert_type(
                x[8 * p:8 * p + 4, :].astype(jnp.bfloat16),
                jnp.uint16).astype(jnp.uint32)
            hi = lax.bitcast_convert_type(
                x[8 * p + 4:8 * p + 8, :].astype(jnp.bfloat16),
                jnp.uint16).astype(jnp.uint32)
            o_ref[j, pl.ds(4 * p, 4), :] = lax.bitcast_convert_type(
                lo | (hi << 16), jnp.int32)


@functools.lru_cache(maxsize=None)
def _make_detile(emb_dim, vocab):
    n_tiles = pl.cdiv(vocab, _LANE)
    m = 1024
    n_jb = pl.cdiv(n_tiles, m)
    return pl.pallas_call(
        _detile_body,
        grid=(n_jb,),
        in_specs=[pl.BlockSpec((emb_dim, _LANE * m), lambda jb: (0, jb))],
        out_specs=pl.BlockSpec((m, emb_dim // 2, _LANE),
                               lambda jb: (jb, 0, 0)),
        out_shape=jax.ShapeDtypeStruct((n_tiles, emb_dim // 2, _LANE),
                                       jnp.int32),
    )


@functools.lru_cache(maxsize=None)
def _make_gather(batch, n_pk, flat_len):
    b_per_w = batch // _NW
    n_q = b_per_w // _LANE
    mesh = plsc.VectorSubcoreMesh(core_axis_name="c", subcore_axis_name="s")
    out_sh = jax.ShapeDtypeStruct((n_pk, batch), jnp.int32)

    def build_flat_idx(idx_v, fidx_v):
        for q in range(n_q):
            for s in range(_LANE // 16):
                r = idx_v[q, pl.ds(s * 16, 16)]
                base = (r >> 7) * (n_pk * _LANE) + (r & (_LANE - 1))
                for c in range(n_pk):
                    fidx_v[c, q, pl.ds(s * 16, 16)] = base + _LANE * c

    @functools.partial(
        pl.kernel,
        mesh=mesh,
        out_type=out_sh,
        scratch_types=[
            pltpu.VMEM((n_q, _LANE), jnp.int32),
            pltpu.VMEM((n_pk, n_q, _LANE), jnp.int32),
            pltpu.VMEM((n_pk, b_per_w), jnp.int32),
            pltpu.SemaphoreType.DMA,
        ],
        compiler_params=pltpu.CompilerParams(use_tc_tiling_on_sc=False),
    )
    def gather_kernel(idx_hbm, tab_hbm, out_hbm, idx_v, fidx_v, rows_v, sem):
        wid = lax.axis_index("s") * _NC + lax.axis_index("c")
        base = wid * b_per_w
        for q in range(n_q):
            pltpu.sync_copy(idx_hbm.at[pl.ds(base + q * _LANE, _LANE)],
                            idx_v.at[q])
        build_flat_idx(idx_v, fidx_v)
        copies = []
        for c in range(n_pk):
            for q in range(n_q):
                copies.append(pltpu.async_copy(
                    tab_hbm.at[fidx_v.at[c, q]],
                    rows_v.at[c, pl.ds(q * _LANE, _LANE)], sem))
        for cp in copies:
            cp.wait()
        pltpu.sync_copy(rows_v, out_hbm.at[:, pl.ds(base, b_per_w)])

    return gather_kernel


def _unpack(x):
    xu = lax.bitcast_convert_type(x, jnp.uint32)
    lo = lax.bitcast_convert_type(xu << 16, jnp.float32)
    hi = lax.bitcast_convert_type(xu & jnp.uint32(0xFFFF0000), jnp.float32)
    return lo, hi


def _mlp_body(ut_ref, vt_ref, w1_ref, b1_ref, w2_ref, b2_ref, w3_ref, b3_ref,
              o_ref):
    n_pk = ut_ref.shape[0]
    ulo, uhi = _unpack(ut_ref[...])
    vlo, vhi = _unpack(vt_ref[...])
    dn = (((0,), (0,)), ((), ()))
    h = lax.dot_general(w1_ref[0:n_pk, :], ulo, dn,
                        preferred_element_type=jnp.float32)
    h = h + lax.dot_general(w1_ref[n_pk:2 * n_pk, :], uhi, dn,
                            preferred_element_type=jnp.float32)
    h = h + lax.dot_general(w1_ref[2 * n_pk:3 * n_pk, :], vlo, dn,
                            preferred_element_type=jnp.float32)
    h = h + lax.dot_general(w1_ref[3 * n_pk:, :], vhi, dn,
                            preferred_element_type=jnp.float32)
    h = jnp.maximum(h + b1_ref[...][:, None], 0.0)
    h = lax.dot_general(w2_ref[...], h, dn,
                        preferred_element_type=jnp.float32)
    h = jnp.maximum(h + b2_ref[...][:, None], 0.0)
    w3 = w3_ref[...]
    o_ref[...] = jnp.sum(h * w3, axis=0) + b3_ref[...]


def kernel(user, item, user_emb, item_emb, W1, b1, W2, b2, W3, b3):
    batch = user.shape[0]
    vocab, emb_dim = user_emb.shape
    n_pk = emb_dim // 2
    detile = _make_detile(emb_dim, vocab)
    u_tab = detile(user_emb.T)
    v_tab = detile(item_emb.T)
    flat_len = u_tab.shape[0] * n_pk * _LANE
    gather = _make_gather(batch, n_pk, flat_len)
    ut_g = gather(user, u_tab.reshape(flat_len))
    vt_g = gather(item, v_tab.reshape(flat_len))
    perm_lo = [8 * (c2 // 4) + c2 % 4 for c2 in range(n_pk)]
    perm = (perm_lo + [p + 4 for p in perm_lo]
            + [p + emb_dim for p in perm_lo]
            + [p + emb_dim + 4 for p in perm_lo])
    w1p = W1[jnp.asarray(perm, dtype=jnp.int32), :]
    return pl.pallas_call(
        _mlp_body,
        out_shape=jax.ShapeDtypeStruct((batch,), jnp.float32),
    )(ut_g, vt_g, w1p, b1, W2, b2, W3, b3)
